# baseline (speedup 1.0000x reference)
import functools

import jax
import jax.numpy as jnp
from jax import lax
from jax.experimental import pallas as pl
from jax.experimental.pallas import tpu as pltpu
from jax.experimental.pallas import tpu_sc as plsc

B = 16384
F = 26
V = 100000
D = 16

NC = 2
NS = 16
FP = F // 2
TABW = 100224
CHUNK = 2048
MASKHI = -65536


def _sc_lin_body(off_hbm, wpack_hbm, out_hbm, tab_v, idx_v, out_v, sem, osem):
    cid = lax.axis_index("c")
    sid = lax.axis_index("s")
    wid = sid * NC + cid

    @pl.when(wid < FP)
    def _():
        start = pl.multiple_of((wid * V // 128) * 128, 128)
        a = pltpu.async_copy(wpack_hbm.at[pl.ds(start, TABW)], tab_v, sem)
        b = pltpu.async_copy(off_hbm.at[pl.ds(wid * B, B)], idx_v, sem)
        a.wait()
        b.wait()

        pend = []
        nburst = B // CHUNK
        for half in range(2):
            if half == 1:
                pltpu.sync_copy(off_hbm.at[pl.ds((wid + FP) * B, B)], idx_v)
            for cc in range(nburst):
                buf = cc % 2
                if len(pend) == 2:
                    pend.pop(0).wait()
                for i in range(CHUNK // 16):
                    iv = idx_v[pl.ds(cc * CHUNK + i * 16, 16)]
                    w = plsc.load_gather(tab_v, [iv])
                    bits = (w << 16) if half == 0 else (w & MASKHI)
                    out_v[buf, pl.ds(i * 16, 16)] = plsc.bitcast(
                        bits, jnp.float32
                    )
                pend.append(
                    pltpu.async_copy(
                        out_v.at[buf],
                        out_hbm.at[
                            pl.ds((wid + half * FP) * B + cc * CHUNK, CHUNK)
                        ],
                        osem,
                    )
                )
        for p in pend:
            p.wait()


def _sc_linear(off, wpack):
    mesh = plsc.VectorSubcoreMesh(core_axis_name="c", subcore_axis_name="s")
    return pl.kernel(
        _sc_lin_body,
        out_type=jax.ShapeDtypeStruct((F * B,), jnp.float32),
        mesh=mesh,
        compiler_params=pltpu.CompilerParams(needs_layout_passes=False),
        scratch_types=[
            pltpu.VMEM((TABW,), jnp.int32),
            pltpu.VMEM((B,), jnp.int32),
            pltpu.VMEM((2, CHUNK), jnp.float32),
            pltpu.SemaphoreType.DMA,
            pltpu.SemaphoreType.DMA,
        ],
    )(off, wpack)


def _tc_inter_body(fe_ref, out_ref):
    acc = fe_ref[0]
    acc2 = acc * acc
    for f in range(1, F):
        v = fe_ref[f]
        acc = acc + v
        acc2 = acc2 + v * v
    out_ref[...] = (acc * acc - acc2) * 0.5


def _tc_interaction(fe_t):
    bt = 8192
    return pl.pallas_call(
        _tc_inter_body,
        grid=(B // bt,),
        in_specs=[pl.BlockSpec((F, D, bt), lambda i: (0, 0, i))],
        out_specs=pl.BlockSpec((D, bt), lambda i: (0, i)),
        out_shape=jax.ShapeDtypeStruct((D, B), jnp.float32),
    )(fe_t)


def kernel(x, feature_emb, w_linear, bias):
    fe_t = feature_emb.transpose(1, 2, 0)
    wb16 = jax.lax.bitcast_convert_type(
        w_linear.astype(jnp.bfloat16), jnp.uint16
    ).astype(jnp.int32)
    wpack = (wb16[:FP] | (wb16[FP:] << 16)).reshape(FP * V)
    delta = ((jnp.arange(F, dtype=jnp.int32) % FP) * V) % 128
    off = (x.T.astype(jnp.int32) + delta[:, None]).reshape(F * B)
    vals = _sc_linear(off, wpack)
    inter_t = _tc_interaction(fe_t)
    lin = functools.reduce(
        lambda a, b: a + b, [vals[f * B : (f + 1) * B] for f in range(F)]
    )
    return (inter_t + (lin + bias[0])[None, :]).T


# speedup vs baseline: 1.3734x; 1.3734x over previous
import functools

import jax
import jax.numpy as jnp
from jax import lax
from jax.experimental import pallas as pl
from jax.experimental.pallas import tpu as pltpu
from jax.experimental.pallas import tpu_sc as plsc

B = 16384
F = 26
V = 100000
D = 16

NC = 2
NS = 16
TABW = 100224
CHUNK = 2048


def _sc_lin_body(off_hbm, wflat_hbm, out_hbm, tab_v, idx_v, out_v, sem, osem):
    cid = lax.axis_index("c")
    sid = lax.axis_index("s")
    wid = sid * NC + cid

    @pl.when(wid < F)
    def _():
        start = pl.multiple_of((wid * V // 128) * 128, 128)
        a = pltpu.async_copy(wflat_hbm.at[pl.ds(start, TABW)], tab_v, sem)
        b = pltpu.async_copy(off_hbm.at[pl.ds(wid * B, B)], idx_v, sem)
        a.wait()
        b.wait()

        pend = []
        for c in range(B // CHUNK):
            buf = c % 2
            if len(pend) == 2:
                pend.pop(0).wait()
            for i in range(CHUNK // 16):
                iv = idx_v[pl.ds(c * CHUNK + i * 16, 16)]
                out_v[buf, pl.ds(i * 16, 16)] = plsc.load_gather(tab_v, [iv])
            pend.append(
                pltpu.async_copy(
                    out_v.at[buf],
                    out_hbm.at[pl.ds(wid * B + c * CHUNK, CHUNK)],
                    osem,
                )
            )
        for p in pend:
            p.wait()


def _sc_linear(off, wflat):
    mesh = plsc.VectorSubcoreMesh(core_axis_name="c", subcore_axis_name="s")
    return pl.kernel(
        _sc_lin_body,
        out_type=jax.ShapeDtypeStruct((F * B,), jnp.float32),
        mesh=mesh,
        compiler_params=pltpu.CompilerParams(needs_layout_passes=False),
        scratch_types=[
            pltpu.VMEM((TABW,), jnp.float32),
            pltpu.VMEM((B,), jnp.int32),
            pltpu.VMEM((2, CHUNK), jnp.float32),
            pltpu.SemaphoreType.DMA,
            pltpu.SemaphoreType.DMA,
        ],
    )(off, wflat)


def _tc_inter_body(fe_ref, out_ref):
    acc = fe_ref[0]
    acc2 = acc * acc
    for f in range(1, F):
        v = fe_ref[f]
        acc = acc + v
        acc2 = acc2 + v * v
    out_ref[...] = (acc * acc - acc2) * 0.5


def _tc_interaction(fe_t):
    bt = 8192
    return pl.pallas_call(
        _tc_inter_body,
        grid=(B // bt,),
        in_specs=[pl.BlockSpec((F, D, bt), lambda i: (0, 0, i))],
        out_specs=pl.BlockSpec((D, bt), lambda i: (0, i)),
        out_shape=jax.ShapeDtypeStruct((D, B), jnp.float32),
    )(fe_t)


def kernel(x, feature_emb, w_linear, bias):
    fe_t = feature_emb.transpose(1, 2, 0)
    wflat = w_linear.reshape(F * V)
    delta = (jnp.arange(F, dtype=jnp.int32) * V) % 128
    off = (x.T.astype(jnp.int32) + delta[:, None]).reshape(F * B)
    vals = _sc_linear(off, wflat)
    inter_t = _tc_interaction(fe_t)
    lin = functools.reduce(
        lambda a, b: a + b, [vals[f * B : (f + 1) * B] for f in range(F)]
    )
    return (inter_t + (lin + bias[0])[None, :]).T


# speedup vs baseline: 1.3737x; 1.0002x over previous
import functools

import jax
import jax.numpy as jnp
from jax import lax
from jax.experimental import pallas as pl
from jax.experimental.pallas import tpu as pltpu
from jax.experimental.pallas import tpu_sc as plsc

B = 16384
F = 26
V = 100000
D = 16

NC = 2
NS = 16
TABW = 100224
CHUNK = 2048


def _sc_lin_body(off_hbm, wflat_hbm, out_hbm, tab_v, idx_v, out_v, sem, osem):
    cid = lax.axis_index("c")
    sid = lax.axis_index("s")
    wid = sid * NC + cid

    @pl.when(wid < F)
    def _():
        start = pl.multiple_of((wid * V // 128) * 128, 128)
        qw = TABW // 4
        stage = [
            pltpu.async_copy(
                wflat_hbm.at[pl.ds(start + q * qw, qw)],
                tab_v.at[pl.ds(q * qw, qw)],
                sem,
            )
            for q in range(4)
        ]
        stage.append(pltpu.async_copy(off_hbm.at[pl.ds(wid * B, B)], idx_v, sem))
        for s in stage:
            s.wait()

        pend = []
        for c in range(B // CHUNK):
            buf = c % 2
            if len(pend) == 2:
                pend.pop(0).wait()
            for i in range(CHUNK // 16):
                iv = idx_v[pl.ds(c * CHUNK + i * 16, 16)]
                out_v[buf, pl.ds(i * 16, 16)] = plsc.load_gather(tab_v, [iv])
            pend.append(
                pltpu.async_copy(
                    out_v.at[buf],
                    out_hbm.at[pl.ds(wid * B + c * CHUNK, CHUNK)],
                    osem,
                )
            )
        for p in pend:
            p.wait()


def _sc_linear(off, wflat):
    mesh = plsc.VectorSubcoreMesh(core_axis_name="c", subcore_axis_name="s")
    return pl.kernel(
        _sc_lin_body,
        out_type=jax.ShapeDtypeStruct((F * B,), jnp.float32),
        mesh=mesh,
        compiler_params=pltpu.CompilerParams(needs_layout_passes=False),
        scratch_types=[
            pltpu.VMEM((TABW,), jnp.float32),
            pltpu.VMEM((B,), jnp.int32),
            pltpu.VMEM((2, CHUNK), jnp.float32),
            pltpu.SemaphoreType.DMA,
            pltpu.SemaphoreType.DMA,
        ],
    )(off, wflat)


def _tc_inter_body(fe_ref, out_ref):
    acc = fe_ref[0]
    acc2 = acc * acc
    for f in range(1, F):
        v = fe_ref[f]
        acc = acc + v
        acc2 = acc2 + v * v
    out_ref[...] = (acc * acc - acc2) * 0.5


def _tc_interaction(fe_t):
    bt = 8192
    return pl.pallas_call(
        _tc_inter_body,
        grid=(B // bt,),
        in_specs=[pl.BlockSpec((F, D, bt), lambda i: (0, 0, i))],
        out_specs=pl.BlockSpec((D, bt), lambda i: (0, i)),
        out_shape=jax.ShapeDtypeStruct((D, B), jnp.float32),
    )(fe_t)


def kernel(x, feature_emb, w_linear, bias):
    fe_t = feature_emb.transpose(1, 2, 0)
    wflat = w_linear.reshape(F * V)
    delta = (jnp.arange(F, dtype=jnp.int32) * V) % 128
    off = (x.T.astype(jnp.int32) + delta[:, None]).reshape(F * B)
    vals = _sc_linear(off, wflat)
    inter_t = _tc_interaction(fe_t)
    lin = functools.reduce(
        lambda a, b: a + b, [vals[f * B : (f + 1) * B] for f in range(F)]
    )
    return (inter_t + (lin + bias[0])[None, :]).T
